# baseline (speedup 1.0000x reference)
import jax
import jax.numpy as jnp
from jax import lax
from jax.experimental import pallas as pl
from jax.experimental.pallas import tpu as pltpu
from jax.experimental.pallas import tpu_sc as plsc

NUM_FEAT_TBL = 16384
G = 4096
FP = 64
D = 256
L = 16
NW = 32
GPW = G // NW
CD = D // L
GC = 2
NCH = GPW // GC
DW = D // 2


def _grouper_sc(table_hbm, idx_hbm, m_hbm, out_hbm, idx_a, m_a, rows0,
                rows1, out_a, sem0, sem1):
    wid = lax.axis_index("s") * 2 + lax.axis_index("c")
    g0 = wid * GPW

    pltpu.sync_copy(idx_hbm.at[pl.ds(g0 * FP, GPW * FP)], idx_a)
    pltpu.sync_copy(m_hbm.at[pl.ds(g0 * L, GPW * L)], m_a)

    rows = (rows0, rows1)
    sems = (sem0, sem1)

    H = FP // 2

    def start(ch, b):
        for j in range(GC):
            t = ch * GC + j
            mt = m_a[pl.ds(t * L, L)][0]
            pltpu.async_copy(
                table_hbm.at[idx_a.at[pl.ds(t * FP, H)]],
                rows[b].at[pl.ds(j * FP, H), :], sems[b])

            @pl.when(mt > H)
            def _(j=j, t=t):
                pltpu.async_copy(
                    table_hbm.at[idx_a.at[pl.ds(t * FP + H, H)]],
                    rows[b].at[pl.ds(j * FP + H, H), :], sems[b])

    def wait(ch, b):
        for j in range(GC):
            t = ch * GC + j
            mt = m_a[pl.ds(t * L, L)][0]
            pltpu.make_async_copy(
                table_hbm.at[idx_a.at[pl.ds(0, H)]],
                rows[b].at[pl.ds(j * FP, H), :], sems[b]).wait()

            @pl.when(mt > H)
            def _(j=j):
                pltpu.make_async_copy(
                    table_hbm.at[idx_a.at[pl.ds(0, H)]],
                    rows[b].at[pl.ds(j * FP + H, H), :], sems[b]).wait()

    def row_terms(rows_b, r):
        terms = []
        for k in range(DW // L):
            w = rows_b[r, pl.ds(k * L, L)]
            lo = jax.lax.bitcast_convert_type(w << 16, jnp.float32)
            hi = jax.lax.bitcast_convert_type(w & jnp.int32(-65536),
                                              jnp.float32)
            terms.append(lo)
            terms.append(hi)
        return terms

    def reduce_chunk(ch, b):
        rows_b = rows[b]
        for j in range(GC):
            t = ch * GC + j
            mt = m_a[pl.ds(t * L, L)][0]

            def fbody(f, a, j=j):
                terms = row_terms(rows_b, j * FP + f)
                return tuple(x + y for x, y in zip(a, terms))

            def fbody2(k, a, j=j):
                f = 1 + 2 * k
                terms = row_terms(rows_b, j * FP + f)
                a = tuple(x + y for x, y in zip(a, terms))
                terms = row_terms(rows_b, j * FP + f + 1)
                return tuple(x + y for x, y in zip(a, terms))

            acc = tuple(row_terms(rows_b, j * FP))
            n2 = (mt - 1) // 2
            acc = lax.fori_loop(0, n2, fbody2, acc)
            acc = lax.fori_loop(1 + 2 * n2, mt, fbody, acc)
            for c in range(CD):
                out_a[pl.ds(t * D + c * L, L)] = acc[c]

    start(0, 0)

    def body(cc, carry):
        c0 = cc * 2
        start(c0 + 1, 1)
        wait(c0, 0)
        reduce_chunk(c0, 0)

        @pl.when(c0 + 2 < NCH)
        def _():
            start(c0 + 2, 0)

        wait(c0 + 1, 1)
        reduce_chunk(c0 + 1, 1)
        return carry

    lax.fori_loop(0, NCH // 2, body, 0)
    pltpu.sync_copy(out_a, out_hbm.at[pl.ds(g0 * D, GPW * D)])


def kernel(in_features, W, grp_edge_feat, edge_to_node, grp_edge_idx_plus,
           grp_num_feat, grp_feat_idx_plus):
    ratio = 1.0 / grp_num_feat.astype(jnp.float32)
    csum = jnp.cumsum(
        jnp.broadcast_to(ratio[:, None], (G, FP)), axis=1)
    hard = csum <= 1.0
    m = jnp.sum(hard, axis=1).astype(jnp.int32)
    m_rep = jnp.repeat(m, L)

    idx_flat = grp_feat_idx_plus.reshape(-1).astype(jnp.int32)

    table_bf = in_features.astype(jnp.bfloat16)
    table_pk = jax.lax.bitcast_convert_type(
        table_bf.reshape(NUM_FEAT_TBL, DW, 2), jnp.int32)

    mesh = plsc.VectorSubcoreMesh(core_axis_name="c", subcore_axis_name="s")
    run = pl.kernel(
        _grouper_sc,
        out_type=jax.ShapeDtypeStruct((G * D,), jnp.float32),
        mesh=mesh,
        scratch_types=[
            pltpu.VMEM((GPW * FP,), jnp.int32),
            pltpu.VMEM((GPW * L,), jnp.int32),
            pltpu.VMEM((GC * FP, DW), jnp.int32),
            pltpu.VMEM((GC * FP, DW), jnp.int32),
            pltpu.VMEM((GPW * D,), jnp.float32),
            pltpu.SemaphoreType.DMA,
            pltpu.SemaphoreType.DMA,
        ],
    )
    out = run(table_pk, idx_flat, m_rep)
    return (out.reshape(G, D // 32, 2, L)
            .transpose(0, 1, 3, 2).reshape(G, D))


# speedup vs baseline: 2.3862x; 2.3862x over previous
import jax
import jax.numpy as jnp
from jax import lax
from jax.experimental import pallas as pl
from jax.experimental.pallas import tpu as pltpu
from jax.experimental.pallas import tpu_sc as plsc

G = 4096
FP = 64
D = 256
L = 16
NW = 32
GPW = G // NW
CD = D // L
GC = 2
NCH = GPW // GC


def _grouper_sc(table_hbm, idx_hbm, m_hbm, out_hbm, idx_a, m_a, rows0,
                rows1, out_a, sem0, sem1):
    wid = lax.axis_index("s") * 2 + lax.axis_index("c")
    g0 = wid * GPW

    pltpu.sync_copy(idx_hbm.at[pl.ds(g0 * FP, GPW * FP)], idx_a)
    pltpu.sync_copy(m_hbm.at[pl.ds(g0 * L, GPW * L)], m_a)

    rows = (rows0, rows1)
    sems = (sem0, sem1)

    H = FP // 2

    def start(ch, b):
        for j in range(GC):
            t = ch * GC + j
            mt = m_a[pl.ds(t * L, L)][0]
            pltpu.async_copy(
                table_hbm.at[idx_a.at[pl.ds(t * FP, H)]],
                rows[b].at[pl.ds(j * FP, H), :], sems[b])

            @pl.when(mt > H)
            def _(j=j, t=t):
                pltpu.async_copy(
                    table_hbm.at[idx_a.at[pl.ds(t * FP + H, H)]],
                    rows[b].at[pl.ds(j * FP + H, H), :], sems[b])

    def wait(ch, b):
        for j in range(GC):
            t = ch * GC + j
            mt = m_a[pl.ds(t * L, L)][0]
            pltpu.make_async_copy(
                table_hbm.at[idx_a.at[pl.ds(0, H)]],
                rows[b].at[pl.ds(j * FP, H), :], sems[b]).wait()

            @pl.when(mt > H)
            def _(j=j):
                pltpu.make_async_copy(
                    table_hbm.at[idx_a.at[pl.ds(0, H)]],
                    rows[b].at[pl.ds(j * FP + H, H), :], sems[b]).wait()

    def reduce_chunk(ch, b):
        rows_b = rows[b]
        for j in range(GC):
            t = ch * GC + j
            mt = m_a[pl.ds(t * L, L)][0]

            def fbody(f, a, j=j):
                a = list(a)
                for c in range(CD):
                    a[c] = a[c] + rows_b[j * FP + f, pl.ds(c * L, L)]
                return tuple(a)

            def fbody2(k, a, j=j):
                a = list(a)
                f = 1 + 2 * k
                for c in range(CD):
                    a[c] = a[c] + rows_b[j * FP + f, pl.ds(c * L, L)]
                for c in range(CD):
                    a[c] = a[c] + rows_b[j * FP + f + 1, pl.ds(c * L, L)]
                return tuple(a)

            acc = tuple(rows_b[j * FP, pl.ds(c * L, L)] for c in range(CD))
            n2 = (mt - 1) // 2
            acc = lax.fori_loop(0, n2, fbody2, acc)
            acc = lax.fori_loop(1 + 2 * n2, mt, fbody, acc)
            for c in range(CD):
                out_a[pl.ds(t * D + c * L, L)] = acc[c]

    start(0, 0)

    def body(cc, carry):
        c0 = cc * 2
        start(c0 + 1, 1)
        wait(c0, 0)
        reduce_chunk(c0, 0)

        @pl.when(c0 + 2 < NCH)
        def _():
            start(c0 + 2, 0)

        wait(c0 + 1, 1)
        reduce_chunk(c0 + 1, 1)
        return carry

    lax.fori_loop(0, NCH // 2, body, 0)
    pltpu.sync_copy(out_a, out_hbm.at[pl.ds(g0 * D, GPW * D)])


def kernel(in_features, W, grp_edge_feat, edge_to_node, grp_edge_idx_plus,
           grp_num_feat, grp_feat_idx_plus):
    ratio = 1.0 / grp_num_feat.astype(jnp.float32)
    csum = jnp.cumsum(
        jnp.broadcast_to(ratio[:, None], (G, FP)), axis=1)
    hard = csum <= 1.0
    m = jnp.sum(hard, axis=1).astype(jnp.int32)
    m_rep = jnp.repeat(m, L)

    idx_flat = grp_feat_idx_plus.reshape(-1).astype(jnp.int32)

    mesh = plsc.VectorSubcoreMesh(core_axis_name="c", subcore_axis_name="s")
    run = pl.kernel(
        _grouper_sc,
        out_type=jax.ShapeDtypeStruct((G * D,), jnp.float32),
        mesh=mesh,
        scratch_types=[
            pltpu.VMEM((GPW * FP,), jnp.int32),
            pltpu.VMEM((GPW * L,), jnp.int32),
            pltpu.VMEM((GC * FP, D), jnp.float32),
            pltpu.VMEM((GC * FP, D), jnp.float32),
            pltpu.VMEM((GPW * D,), jnp.float32),
            pltpu.SemaphoreType.DMA,
            pltpu.SemaphoreType.DMA,
        ],
    )
    return run(in_features, idx_flat, m_rep).reshape(G, D)
